# baseline (speedup 1.0000x reference)
import jax
import jax.numpy as jnp
from jax.experimental import pallas as pl

B_BLK = 128


def _add_pe_kernel(x_ref, pe_ref, o_ref):
    o_ref[...] = x_ref[...] + pe_ref[...]


def kernel(x, pos_emb):
    B, S, D = x.shape
    x2 = x.reshape(B, S * D)
    pe = pos_emb[:S].reshape(1, S * D)
    out = pl.pallas_call(
        _add_pe_kernel,
        grid=(B // B_BLK,),
        in_specs=[
            pl.BlockSpec((B_BLK, S * D), lambda i: (i, 0)),
            pl.BlockSpec((1, S * D), lambda i: (0, 0)),
        ],
        out_specs=pl.BlockSpec((B_BLK, S * D), lambda i: (i, 0)),
        out_shape=jax.ShapeDtypeStruct((B, S * D), x.dtype),
    )(x2, pe)
    return out.reshape(B, S, D)
